# Initial kernel scaffold; baseline (speedup 1.0000x reference)
import jax
import jax.numpy as jnp
from jax.experimental import pallas as pl


def kernel(pred_logits, target_classes, pred_count, counter_target, pred_captions, target_caption):
    raise NotImplementedError("write your pallas kernel here")


# speedup vs baseline: 3.2305x; 3.2305x over previous
import functools

import jax
import jax.numpy as jnp
from jax.experimental import pallas as pl

NUM_CLASSES = 100
EOS_COEF = 0.1
PAD_IDX = 1
SMOOTHING = 0.7
_CCR11 = [0.0, 0.0, 0.193425917, 0.412129084, 0.188929963, 0.0781296833,
          0.0509541413, 0.0312718553, 0.018483365, 0.0083924468, 0.00659406534]

_V = 10000
_ROWS = 64 * 30
_BR = 128
_GRID = _ROWS // _BR
_NL = 64 * 100


def _loss_kernel(cap_ref, tcap_ref, logit_ref, tcls_ref, pc_ref, ct_ref, out_ref):
    i = pl.program_id(0)
    eps = SMOOTHING / (_V - 2)
    c_row = SMOOTHING * jnp.log(eps) + (1.0 - SMOOTHING) * jnp.log(1.0 - SMOOTHING)

    @pl.when(i == 0)
    def _small_losses():
        x = logit_ref[...]
        tc = tcls_ref[...]
        cid = jax.lax.broadcasted_iota(jnp.int32, x.shape, 1)
        m = jnp.max(x, axis=1, keepdims=True)
        lse = jnp.log(jnp.sum(jnp.exp(x - m), axis=1, keepdims=True)) + m
        xt = jnp.sum(jnp.where(cid == tc, x, 0.0), axis=1, keepdims=True)
        w = jnp.where(tc == NUM_CLASSES, EOS_COEF, 1.0)
        loss_ce = jnp.sum(w * (lse - xt), keepdims=True) / jnp.sum(w)

        pc = pc_ref[...]
        ct = ct_ref[...]
        j = jax.lax.broadcasted_iota(jnp.int32, pc.shape, 1)
        onehot = (j == ct)
        diff = (j - ct).astype(jnp.float32)
        gmask = jnp.exp(-diff * diff / 8.0)
        tgt = onehot.astype(jnp.float32)
        bce = (jnp.maximum(pc, 0.0) - pc * tgt
               + jnp.log1p(jnp.exp(-jnp.abs(pc))))
        coef = jnp.where(onehot, 1.0, 1.0 - gmask)
        wccr = jnp.zeros(pc.shape, jnp.float32)
        for k, v in enumerate(_CCR11):
            wccr = jnp.where(j == k, 1.0 - v, wccr)
        loss_counter = (jnp.sum(bce * wccr * coef, keepdims=True)
                        / (pc.shape[0] * pc.shape[1]))

        out_ref[...] = loss_ce + loss_counter

    lp = jnp.log(cap_ref[...])
    t = tcap_ref[...]
    vid = jax.lax.broadcasted_iota(jnp.int32, lp.shape, 1)
    s_all = jnp.sum(lp, axis=1, keepdims=True)
    g = jnp.sum(jnp.where(vid == t, lp, 0.0), axis=1, keepdims=True)
    p = lp[:, PAD_IDX:PAD_IDX + 1]
    kl = c_row - eps * (s_all - p - g) - (1.0 - SMOOTHING) * g
    kl = jnp.where(t == PAD_IDX, 0.0, kl)
    out_ref[...] += jnp.sum(kl, keepdims=True)


@functools.partial(jax.jit, static_argnames=())
def kernel(pred_logits, target_classes, pred_count, counter_target,
           pred_captions, target_caption):
    cap = pred_captions.reshape(_ROWS, _V)
    tcap = target_caption.reshape(_ROWS, 1).astype(jnp.int32)
    logits = pred_logits.reshape(_NL, NUM_CLASSES + 1)
    tcls = target_classes.reshape(_NL, 1).astype(jnp.int32)
    ct = counter_target.reshape(64, 1).astype(jnp.int32)

    out = pl.pallas_call(
        _loss_kernel,
        grid=(_GRID,),
        in_specs=[
            pl.BlockSpec((_BR, _V), lambda i: (i, 0)),
            pl.BlockSpec((_BR, 1), lambda i: (i, 0)),
            pl.BlockSpec((_NL, NUM_CLASSES + 1), lambda i: (0, 0)),
            pl.BlockSpec((_NL, 1), lambda i: (0, 0)),
            pl.BlockSpec((64, 11), lambda i: (0, 0)),
            pl.BlockSpec((64, 1), lambda i: (0, 0)),
        ],
        out_specs=pl.BlockSpec((1, 1), lambda i: (0, 0)),
        out_shape=jax.ShapeDtypeStruct((1, 1), jnp.float32),
    )(cap, tcap, logits, tcls, pred_count, ct)
    return out[0, 0]
